# Initial kernel scaffold; baseline (speedup 1.0000x reference)
import jax
import jax.numpy as jnp
from jax.experimental import pallas as pl


def kernel(x, edge, conv1_w, conv2_w, w0, b0, w1, b1):
    raise NotImplementedError("write your pallas kernel here")


# speedup vs baseline: 1.4470x; 1.4470x over previous
import functools

import jax
import jax.numpy as jnp
from jax.experimental import pallas as pl
from jax.experimental.pallas import tpu as pltpu


def _fused_kernel(eflat_ref, x_ref, mask1_ref, mask2_ref,
                  wc1_ref, wc2a_ref, w0_ref, b0_ref, w1_ref, b1_ref,
                  out_ref, *, S):
    f32 = jnp.float32
    bt = eflat_ref.shape[0]
    SS = eflat_ref.shape[1]
    Ce = eflat_ref.shape[2]
    C0 = wc1_ref.shape[1]
    F = wc2a_ref.shape[1]

    wc1 = wc1_ref[...]
    wc2a = wc2a_ref[...]
    w0 = w0_ref[...]
    w1 = w1_ref[...]
    b0 = b0_ref[...]
    b1 = b1_ref[...]
    mask1 = mask1_ref[...]
    mask2 = mask2_ref[...]

    pad = S + 1
    shifts = [(dy - 1) * S + (dx - 1) for dy in range(3) for dx in range(3)]

    def im2col(flat, nch):
        ext = jnp.concatenate(
            [jnp.zeros((pad, nch), f32), flat, jnp.zeros((pad, nch), f32)],
            axis=0)
        return jnp.concatenate(
            [ext[pad + d:pad + d + SS, :] for d in shifts], axis=-1)

    for b in range(bt):
        cols1 = im2col(eflat_ref[b], Ce) * mask1
        e1 = jnp.dot(cols1, wc1, preferred_element_type=f32)
        cols2 = im2col(e1, C0) * mask2
        e = jnp.dot(cols2, wc2a, preferred_element_type=f32)
        e3 = e.reshape(S, S, F)

        s0 = jnp.dot(x_ref[b], w0, preferred_element_type=f32)
        out0 = jnp.sum(e3 * s0[None, :, :], axis=1) + b0
        s1 = jnp.dot(out0, w1, preferred_element_type=f32)
        out1 = jnp.sum(e3 * s1[None, :, :], axis=1) + b1
        out_ref[b] = out1.astype(out_ref.dtype)


@functools.partial(jax.jit, static_argnames=("batch_tile",))
def _run(x, edge, conv1_w, conv2_w, w0, b0, w1, b1, batch_tile=8):
    f32 = jnp.float32
    B, S, Fn = x.shape
    Ce = edge.shape[-1]
    C0 = conv1_w.shape[0]
    C1 = conv2_w.shape[0]
    F = C0 + C1
    SS = S * S
    bt = batch_tile

    xf = x.astype(f32)
    eflat = edge.astype(f32).reshape(B, SS, Ce)

    wc1 = jnp.transpose(conv1_w, (2, 3, 1, 0)).reshape(9 * Ce, C0).astype(f32)
    wc2 = jnp.transpose(conv2_w, (2, 3, 1, 0)).reshape(9 * C0, C1).astype(f32)
    eye_center = jnp.zeros((9 * C0, C0), f32).at[4 * C0 + jnp.arange(C0),
                                                 jnp.arange(C0)].set(1.0)
    wc2a = jnp.concatenate([eye_center, wc2], axis=1)

    w0f = w0.astype(f32)
    w1f = w1.astype(f32)
    b0f = b0.reshape(1, F).astype(f32)
    b1f = b1.reshape(1, F).astype(f32)

    rr = jnp.arange(SS)
    ii, jj = rr // S, rr % S
    tmask = jnp.stack(
        [((ii + dy - 1 >= 0) & (ii + dy - 1 < S) &
          (jj + dx - 1 >= 0) & (jj + dx - 1 < S))
         for dy in range(3) for dx in range(3)], axis=-1).astype(f32)
    mask1 = jnp.repeat(tmask, Ce, axis=1)
    mask2 = jnp.repeat(tmask, C0, axis=1)

    def const_spec(shape):
        z = (0,) * len(shape)
        return pl.BlockSpec(shape, lambda g, _z=z: _z)

    flops = 2 * B * (SS * (9 * Ce) * C0 + SS * (9 * C0) * F
                     + S * Fn * F + S * F * F + 2 * SS * F)
    bytes_accessed = 4 * (eflat.size + xf.size + mask1.size + mask2.size
                          + wc1.size + wc2a.size + w0f.size + w1f.size
                          + b0f.size + b1f.size + B * S * F)

    return pl.pallas_call(
        functools.partial(_fused_kernel, S=S),
        grid=(B // bt,),
        in_specs=[
            pl.BlockSpec((bt, SS, Ce), lambda g: (g, 0, 0)),
            pl.BlockSpec((bt, S, Fn), lambda g: (g, 0, 0)),
            const_spec((SS, 9 * Ce)),
            const_spec((SS, 9 * C0)),
            const_spec((9 * Ce, C0)),
            const_spec((9 * C0, F)),
            const_spec((Fn, F)), const_spec((1, F)),
            const_spec((F, F)), const_spec((1, F)),
        ],
        out_specs=pl.BlockSpec((bt, S, F), lambda g: (g, 0, 0)),
        out_shape=jax.ShapeDtypeStruct((B, S, F), f32),
        compiler_params=pltpu.CompilerParams(dimension_semantics=("parallel",)),
        cost_estimate=pl.CostEstimate(flops=flops, transcendentals=0,
                                      bytes_accessed=bytes_accessed),
    )(eflat, xf, mask1, mask2, wc1, wc2a, w0f, b0f, w1f, b1f)


def kernel(x, edge, conv1_w, conv2_w, w0, b0, w1, b1):
    return _run(x, edge, conv1_w, conv2_w, w0, b0, w1, b1)
